# baseline (speedup 1.0000x reference)
import functools

import jax
import jax.numpy as jnp
from jax import lax
from jax.experimental import pallas as pl
from jax.experimental.pallas import tpu as pltpu
from jax.experimental.pallas import tpu_sc as plsc

EMBED_D = 32
LANES = 16
BT_PER_SB = 4
SB_IDX = BT_PER_SB * 128
NBUF = 2


def _emb_body(table_hbm, idxt_hbm, out_hbm, idx_v, rows_v, tiles_v, gsem, osem):
    info = plsc.get_sparse_core_info()
    nc = info.num_cores
    nw = nc * info.num_subcores
    wid = lax.axis_index("s") * nc + lax.axis_index("c")

    n_seq = idxt_hbm.shape[0]
    n_batch = idxt_hbm.shape[1]
    n_sb = n_batch // SB_IDX
    total_sb = n_seq * n_sb
    per_w = total_sb // nw

    row_iota = lax.iota(jnp.int32, LANES)

    def fetch(sb, p):
        s = sb // n_sb
        col0 = (sb % n_sb) * SB_IDX
        pltpu.sync_copy(idxt_hbm.at[s, pl.ds(col0, SB_IDX)], idx_v.at[p])
        pltpu.async_copy(table_hbm.at[idx_v.at[p]], rows_v.at[p], gsem.at[p])

    def drain_gather(p):
        pltpu.make_async_copy(
            table_hbm.at[idx_v.at[p]], rows_v.at[p], gsem.at[p]
        ).wait()

    def out_slices(sb, p):
        s = sb // n_sb
        bt0 = (sb % n_sb) * BT_PER_SB
        return [
            (
                tiles_v.at[p, pl.ds(tr * BT_PER_SB * 1024, BT_PER_SB * 1024)],
                out_hbm.at[pl.ds((((s * 4 + tr) * (n_batch // 128)) + bt0) * 1024,
                                 BT_PER_SB * 1024)],
            )
            for tr in range(EMBED_D // 8)
        ]

    pat0 = (row_iota // 8) * (BT_PER_SB * 1024) + (row_iota % 8) * 128
    pat1 = pat0 + 2 * (BT_PER_SB * 1024)

    def process(sb, p):
        @plsc.parallel_loop(0, SB_IDX, unroll=8)
        def _(t):
            base = jnp.broadcast_to((t // 128) * 1024 + (t % 128), (LANES,))
            v0 = rows_v[p, t, pl.ds(0, LANES)]
            v1 = rows_v[p, t, pl.ds(LANES, LANES)]
            plsc.store_scatter(tiles_v.at[p], [pat0 + base], v0)
            plsc.store_scatter(tiles_v.at[p], [pat1 + base], v1)

        for src, dst in out_slices(sb, p):
            pltpu.async_copy(src, dst, osem.at[p])

    def drain_out(sb, p):
        for src, dst in out_slices(sb, p):
            pltpu.make_async_copy(src, dst, osem.at[p]).wait()

    first_sb = wid * per_w
    fetch(first_sb, 0)

    def body(k2, carry):
        for p in range(NBUF):
            sb = first_sb + k2 * NBUF + p
            drain_gather(p)

            @pl.when(sb + 1 < first_sb + per_w)
            def _():
                fetch(sb + 1, (p + 1) % NBUF)

            @pl.when(sb - NBUF >= first_sb)
            def _():
                drain_out(sb - NBUF, p)

            process(sb, p)
        return carry

    lax.fori_loop(0, per_w // NBUF, body, 0)
    for p in range(NBUF):
        drain_out(first_sb + per_w - NBUF + p, p)


def kernel(inputs, embeddings):
    b, s = inputs.shape
    idxt = jnp.transpose(inputs).astype(jnp.int32)
    emb = pl.kernel(
        _emb_body,
        mesh=plsc.VectorSubcoreMesh(core_axis_name="c", subcore_axis_name="s"),
        out_type=jax.ShapeDtypeStruct((s * EMBED_D * b,), jnp.float32),
        scratch_types=[
            pltpu.VMEM((NBUF, SB_IDX), jnp.int32),
            pltpu.VMEM((NBUF, SB_IDX, EMBED_D), jnp.float32),
            pltpu.VMEM((NBUF, EMBED_D // 8 * BT_PER_SB * 1024), jnp.float32),
            pltpu.SemaphoreType.DMA((NBUF,)),
            pltpu.SemaphoreType.DMA((NBUF,)),
        ],
        compiler_params=pltpu.CompilerParams(
            use_tc_tiling_on_sc=False, needs_layout_passes=False
        ),
    )
    out5 = emb(embeddings, idxt).reshape(s, EMBED_D // 8, b // 128, 8, 128)
    out = jnp.transpose(out5, (2, 4, 0, 1, 3)).reshape(b, s, EMBED_D)
    return out


# speedup vs baseline: 1.5320x; 1.5320x over previous
import functools

import jax
import jax.numpy as jnp
from jax import lax
from jax.experimental import pallas as pl
from jax.experimental.pallas import tpu as pltpu
from jax.experimental.pallas import tpu_sc as plsc

EMBED_D = 32
LANES = 16
BT_PER_SB = 4
SB_IDX = BT_PER_SB * 128
NBUF = 2
LPAD = 129
J_STRIDE = 8 * LPAD
TR_STRIDE = BT_PER_SB * J_STRIDE
TILES_W = (EMBED_D // 8) * TR_STRIDE


def _emb_body(table_hbm, idxt_hbm, out_hbm, idx_v, rows_v, tiles_v, gsem, osem):
    info = plsc.get_sparse_core_info()
    nc = info.num_cores
    nw = nc * info.num_subcores
    wid = lax.axis_index("s") * nc + lax.axis_index("c")

    n_seq = idxt_hbm.shape[0]
    n_batch = idxt_hbm.shape[1]
    n_sb = n_batch // SB_IDX
    total_sb = n_seq * n_sb
    per_w = total_sb // nw

    row_iota = lax.iota(jnp.int32, LANES)

    def fetch(sb, p):
        s = sb // n_sb
        col0 = (sb % n_sb) * SB_IDX
        pltpu.sync_copy(idxt_hbm.at[s, pl.ds(col0, SB_IDX)], idx_v.at[p])
        pltpu.async_copy(table_hbm.at[idx_v.at[p]], rows_v.at[p], gsem.at[p])

    def drain_gather(p):
        pltpu.make_async_copy(
            table_hbm.at[idx_v.at[p]], rows_v.at[p], gsem.at[p]
        ).wait()

    def out_slices(sb, p):
        s = sb // n_sb
        bt0 = (sb % n_sb) * BT_PER_SB
        res = []
        for tr in range(EMBED_D // 8):
            for j in range(BT_PER_SB):
                src = tiles_v.at[p, pl.ds((tr * BT_PER_SB + j) * 8, 8), pl.ds(0, 128)]
                dst = out_hbm.at[(s * 4 + tr) * (n_batch // 128) + bt0 + j]
                res.append((src, dst))
        return res

    patr0 = (row_iota // 8) * (BT_PER_SB * 8) + (row_iota % 8)
    patr1 = patr0 + 2 * (BT_PER_SB * 8)

    def process(sb, p):
        @plsc.parallel_loop(0, SB_IDX, unroll=8)
        def _(t):
            rbase = jnp.broadcast_to((t // 128) * 8, (LANES,))
            cols = jnp.broadcast_to(t % 128, (LANES,))
            v0 = rows_v[p, t, pl.ds(0, LANES)]
            v1 = rows_v[p, t, pl.ds(LANES, LANES)]
            plsc.store_scatter(tiles_v.at[p], [patr0 + rbase, cols], v0)
            plsc.store_scatter(tiles_v.at[p], [patr1 + rbase, cols], v1)

        for src, dst in out_slices(sb, p):
            pltpu.async_copy(src, dst, osem.at[p])

    def drain_out(sb, p):
        for src, dst in out_slices(sb, p):
            pltpu.make_async_copy(src, dst, osem.at[p]).wait()

    first_sb = wid * per_w
    fetch(first_sb, 0)

    def body(k2, carry):
        for p in range(NBUF):
            sb = first_sb + k2 * NBUF + p
            drain_gather(p)

            @pl.when(sb + 1 < first_sb + per_w)
            def _():
                fetch(sb + 1, (p + 1) % NBUF)

            @pl.when(sb - NBUF >= first_sb)
            def _():
                drain_out(sb - NBUF, p)

            process(sb, p)
        return carry

    lax.fori_loop(0, per_w // NBUF, body, 0)
    for p in range(NBUF):
        drain_out(first_sb + per_w - NBUF + p, p)


def kernel(inputs, embeddings):
    b, s = inputs.shape
    idxt = jnp.transpose(inputs).astype(jnp.int32)
    emb = pl.kernel(
        _emb_body,
        mesh=plsc.VectorSubcoreMesh(core_axis_name="c", subcore_axis_name="s"),
        out_type=jax.ShapeDtypeStruct((s * (EMBED_D // 8) * (b // 128), 8, 128), jnp.float32),
        scratch_types=[
            pltpu.VMEM((NBUF, SB_IDX), jnp.int32),
            pltpu.VMEM((NBUF, SB_IDX, EMBED_D), jnp.float32),
            pltpu.VMEM((NBUF, (EMBED_D // 8) * BT_PER_SB * 8, LPAD), jnp.float32),
            pltpu.SemaphoreType.DMA((NBUF,)),
            pltpu.SemaphoreType.DMA((NBUF,)),
        ],
        compiler_params=pltpu.CompilerParams(
            use_tc_tiling_on_sc=False, needs_layout_passes=False
        ),
    )
    out5 = emb(embeddings, idxt).reshape(s, EMBED_D // 8, b // 128, 8, 128)
    out = jnp.transpose(out5, (2, 4, 0, 1, 3)).reshape(b, s, EMBED_D)
    return out
